# baseline (device time: 10778 ns/iter reference)
import jax
import jax.numpy as jnp
from jax import lax
from jax.experimental import pallas as pl
from jax.experimental.pallas import tpu as pltpu

N_X = 2


def kernel(x):
    m, n = x.shape
    half = n // N_X
    hm = m // 2

    def body(x_ref, out_ref, sems):
        my_x = lax.axis_index("x")
        my_y = lax.axis_index("y")
        other = 1 - my_x

        out_ref[pl.ds(my_x * m, m), :] = x_ref[:, pl.ds(my_x * half, half)]

        barrier_sem = pltpu.get_barrier_semaphore()
        pl.semaphore_signal(
            barrier_sem, inc=1,
            device_id=(other, my_y), device_id_type=pl.DeviceIdType.MESH,
        )
        pl.semaphore_signal(
            barrier_sem, inc=1,
            device_id=(my_x, 1 - my_y), device_id_type=pl.DeviceIdType.MESH,
        )
        pl.semaphore_wait(barrier_sem, 2)

        rdma_a = pltpu.make_async_remote_copy(
            src_ref=x_ref.at[pl.ds(my_y * hm, hm), pl.ds(other * half, half)],
            dst_ref=out_ref.at[pl.ds(my_x * m + my_y * hm, hm), :],
            send_sem=sems.at[0],
            recv_sem=sems.at[1],
            device_id=(other, my_y),
            device_id_type=pl.DeviceIdType.MESH,
        )
        rdma_a.start()
        rdma_a.wait()

        rdma_b = pltpu.make_async_remote_copy(
            src_ref=out_ref.at[pl.ds(other * m + my_y * hm, hm), :],
            dst_ref=out_ref.at[pl.ds(other * m + my_y * hm, hm), :],
            send_sem=sems.at[2],
            recv_sem=sems.at[3],
            device_id=(my_x, 1 - my_y),
            device_id_type=pl.DeviceIdType.MESH,
        )
        rdma_b.start()
        rdma_b.wait()

    return pl.pallas_call(
        body,
        out_shape=jax.ShapeDtypeStruct((N_X * m, half), x.dtype),
        in_specs=[pl.BlockSpec(memory_space=pltpu.VMEM)],
        out_specs=pl.BlockSpec(memory_space=pltpu.VMEM),
        scratch_shapes=[
            pltpu.SemaphoreType.DMA((4,)),
        ],
        compiler_params=pltpu.CompilerParams(collective_id=0),
    )(x)
